# baseline (speedup 1.0000x reference)
import functools

import jax
import jax.numpy as jnp
from jax import lax
from jax.experimental import pallas as pl
from jax.experimental.pallas import tpu as pltpu
from jax.experimental.pallas import tpu_sc as plsc

NC = 2
NS = 16
NW = NC * NS
LANES = 16
SUB0 = 128


def _build_sc_call(b_dim, l_dim, h_dim):
    assert h_dim == LANES
    assert b_dim % NW == 0 and l_dim % 8 == 0 and SUB0 % 8 == 0
    rows_per_w = b_dim // NW
    sub1 = l_dim - SUB0
    full_groups = (l_dim // LANES) * LANES
    mesh = plsc.VectorSubcoreMesh(core_axis_name="c", subcore_axis_name="s")

    @functools.partial(
        pl.kernel,
        mesh=mesh,
        out_type=jax.ShapeDtypeStruct((b_dim, l_dim, h_dim), jnp.float32),
        compiler_params=pltpu.CompilerParams(
            needs_layout_passes=False, use_tc_tiling_on_sc=False),
        scratch_types=[
            pltpu.VMEM((rows_per_w, l_dim), jnp.int32),
            pltpu.VMEM((l_dim, LANES), jnp.float32),
            pltpu.VMEM((l_dim, LANES), jnp.float32),
            pltpu.VMEM((l_dim, LANES), jnp.float32),
            pltpu.VMEM((l_dim, LANES), jnp.float32),
            pltpu.VMEM((LANES, LANES), jnp.float32),
            pltpu.SemaphoreType.DMA,
            pltpu.SemaphoreType.DMA,
            pltpu.SemaphoreType.DMA,
            pltpu.SemaphoreType.DMA,
        ],
    )
    def sc_encode(x_hbm, emb_hbm, pos_hbm, out_hbm, idx_v, e_buf0, e_buf1,
                  o_buf0, o_buf1, pos_v, gsem0, gsem1, ssem0, ssem1):
        wid = lax.axis_index("s") * NC + lax.axis_index("c")
        e_bufs, o_bufs = (e_buf0, e_buf1), (o_buf0, o_buf1)
        gsems, ssems = (gsem0, gsem1), (ssem0, ssem1)
        row_base = wid * rows_per_w
        pltpu.sync_copy(x_hbm.at[pl.ds(row_base, rows_per_w)], idx_v)
        pltpu.sync_copy(pos_hbm, pos_v)
        riota = lax.iota(jnp.int32, LANES)
        jvs = [(riota + c) & (LANES - 1) for c in range(LANES)]
        sixteen = jnp.full((LANES,), LANES, jnp.int32)
        nbuf = 2

        def issue_gather(r, b):
            pltpu.async_copy(
                emb_hbm.at[idx_v.at[r, pl.ds(0, SUB0)]],
                e_bufs[b].at[pl.ds(0, SUB0)], gsems[b])
            pltpu.async_copy(
                emb_hbm.at[idx_v.at[r, pl.ds(SUB0, sub1)]],
                e_bufs[b].at[pl.ds(SUB0, sub1)], gsems[b])

        def wait_gather(r, b):
            pltpu.make_async_copy(
                emb_hbm.at[idx_v.at[r, pl.ds(0, SUB0)]],
                e_bufs[b].at[pl.ds(0, SUB0)], gsems[b]).wait()
            pltpu.make_async_copy(
                emb_hbm.at[idx_v.at[r, pl.ds(SUB0, sub1)]],
                e_bufs[b].at[pl.ds(SUB0, sub1)], gsems[b]).wait()

        for b in range(nbuf):
            issue_gather(b, b)

        def row_body(g, carry):
            for b in range(nbuf):
                r = g * nbuf + b
                wait_gather(r, b)

                @pl.when(g > 0)
                def _():
                    pltpu.make_async_copy(
                        o_bufs[b], out_hbm.at[row_base + r - nbuf],
                        ssems[b]).wait()

                e_buf, o_buf = e_bufs[b], o_bufs[b]

                def do_group(base):
                    rows = riota + base
                    vs = [plsc.load_gather(e_buf, [rows, jv]) for jv in jvs]
                    m = vs[0]
                    for c in range(1, LANES):
                        m = jnp.maximum(m, vs[c])
                    amax = sixteen
                    for c in range(LANES):
                        amax = jnp.minimum(
                            amax, jnp.where(vs[c] == m, jvs[c], sixteen))
                    for c in range(LANES):
                        p = plsc.load_gather(pos_v, [amax, jvs[c]])
                        plsc.store_scatter(o_buf, [rows, jvs[c]], vs[c] + p)

                @plsc.parallel_loop(0, full_groups, step=LANES, unroll=2)
                def _(base):
                    do_group(base)

                if l_dim % LANES:
                    do_group(l_dim - LANES)

                pltpu.async_copy(o_buf, out_hbm.at[row_base + r], ssems[b])

                @pl.when(r + nbuf < rows_per_w)
                def _():
                    issue_gather(r + nbuf, b)
            return carry

        lax.fori_loop(0, rows_per_w // nbuf, row_body, 0)

        for b in range(nbuf):
            r = rows_per_w - nbuf + b
            pltpu.make_async_copy(
                o_bufs[b], out_hbm.at[row_base + r], ssems[b]).wait()

    return sc_encode


def kernel(x, emb_table, pos_table):
    b_dim, l_dim = x.shape
    _, h_dim = emb_table.shape
    call = _build_sc_call(b_dim, l_dim, h_dim)
    return call(x.astype(jnp.int32), emb_table, pos_table)


# speedup vs baseline: 1.9401x; 1.9401x over previous
import functools

import jax
import jax.numpy as jnp
from jax import lax
from jax.experimental import pallas as pl
from jax.experimental.pallas import tpu as pltpu
from jax.experimental.pallas import tpu_sc as plsc

NC = 2
NS = 16
NW = NC * NS
LANES = 16
BBLK = 128


def _build_sc_call(b_dim, l_dim, h_dim):
    assert h_dim == LANES and b_dim == NW * BBLK
    n_hb = h_dim // 8
    n_bt = b_dim // BBLK
    mesh = plsc.VectorSubcoreMesh(core_axis_name="c", subcore_axis_name="s")

    @functools.partial(
        pl.kernel,
        mesh=mesh,
        out_type=jax.ShapeDtypeStruct((l_dim * n_hb * n_bt, 8 * BBLK),
                                      jnp.float32),
        compiler_params=pltpu.CompilerParams(
            needs_layout_passes=False, use_tc_tiling_on_sc=False),
        scratch_types=[
            pltpu.VMEM((l_dim, BBLK), jnp.int32),
            pltpu.VMEM((BBLK, LANES), jnp.float32),
            pltpu.VMEM((BBLK, LANES), jnp.float32),
            pltpu.VMEM((n_hb, 8 * BBLK), jnp.float32),
            pltpu.VMEM((n_hb, 8 * BBLK), jnp.float32),
            pltpu.VMEM((LANES, LANES), jnp.float32),
            pltpu.SemaphoreType.DMA,
            pltpu.SemaphoreType.DMA,
            pltpu.SemaphoreType.DMA,
            pltpu.SemaphoreType.DMA,
        ],
    )
    def sc_encode(xt_hbm, emb_hbm, pos_hbm, out_hbm, idx_v, e_buf0, e_buf1,
                  o_buf0, o_buf1, pos_v, gsem0, gsem1, ssem0, ssem1):
        wid = lax.axis_index("s") * NC + lax.axis_index("c")
        e_bufs, o_bufs = (e_buf0, e_buf1), (o_buf0, o_buf1)
        gsems, ssems = (gsem0, gsem1), (ssem0, ssem1)
        pltpu.sync_copy(xt_hbm.at[:, pl.ds(wid * BBLK, BBLK)], idx_v)
        pltpu.sync_copy(pos_hbm, pos_v)
        riota = lax.iota(jnp.int32, LANES)
        jvs = [(riota + c) & (LANES - 1) for c in range(LANES)]
        hbs = [jv >> 3 for jv in jvs]
        obase = [(jv & 7) << 7 for jv in jvs]
        sixteen = jnp.full((LANES,), LANES, jnp.int32)
        nbuf = 2

        def gather_copy(l, b):
            return pltpu.make_async_copy(
                emb_hbm.at[idx_v.at[l]], e_bufs[b], gsems[b])

        def store_copies(l, b):
            return [
                pltpu.make_async_copy(
                    o_bufs[b].at[hb],
                    out_hbm.at[(l * n_hb + hb) * n_bt + wid], ssems[b])
                for hb in range(n_hb)
            ]

        for b in range(nbuf):
            gather_copy(b, b).start()

        def l_body(g, carry):
            for b in range(nbuf):
                l = g * nbuf + b
                gather_copy(l, b).wait()

                @pl.when(g > 0)
                def _():
                    for cp in store_copies(l - nbuf, b):
                        cp.wait()

                e_buf, o_buf = e_bufs[b], o_bufs[b]

                @plsc.parallel_loop(0, BBLK, step=LANES, unroll=2)
                def _(base):
                    rows = riota + base
                    vs = [plsc.load_gather(e_buf, [rows, jv]) for jv in jvs]
                    m = vs[0]
                    for c in range(1, LANES):
                        m = jnp.maximum(m, vs[c])
                    amax = sixteen
                    for c in range(LANES):
                        amax = jnp.minimum(
                            amax, jnp.where(vs[c] == m, jvs[c], sixteen))
                    for c in range(LANES):
                        p = plsc.load_gather(pos_v, [amax, jvs[c]])
                        plsc.store_scatter(
                            o_buf, [hbs[c], obase[c] + rows], vs[c] + p)

                for cp in store_copies(l, b):
                    cp.start()

                @pl.when(l + nbuf < l_dim)
                def _():
                    gather_copy(l + nbuf, b).start()
            return carry

        lax.fori_loop(0, l_dim // nbuf, l_body, 0)

        for b in range(nbuf):
            for cp in store_copies(l_dim - nbuf + b, b):
                cp.wait()

    return sc_encode


def kernel(x, emb_table, pos_table):
    b_dim, l_dim = x.shape
    _, h_dim = emb_table.shape
    call = _build_sc_call(b_dim, l_dim, h_dim)
    out = call(x.T.astype(jnp.int32), emb_table, pos_table)
    n_hb, n_bt = h_dim // 8, b_dim // BBLK
    out = out.reshape(l_dim, n_hb, n_bt, 8, BBLK)
    return out.transpose(2, 4, 0, 1, 3).reshape(b_dim, l_dim, h_dim)


# speedup vs baseline: 2.4128x; 1.2436x over previous
import functools

import jax
import jax.numpy as jnp
from jax import lax
from jax.experimental import pallas as pl
from jax.experimental.pallas import tpu as pltpu
from jax.experimental.pallas import tpu_sc as plsc

NC = 2
NS = 16
NW = NC * NS
LANES = 16
BBLK = 128


def _build_sc_call(b_dim, l_dim, h_dim):
    assert h_dim == LANES and b_dim == NW * BBLK
    n_hb = h_dim // 8
    n_bt = b_dim // BBLK
    mesh = plsc.VectorSubcoreMesh(core_axis_name="c", subcore_axis_name="s")

    @functools.partial(
        pl.kernel,
        mesh=mesh,
        out_type=jax.ShapeDtypeStruct((l_dim * n_hb * n_bt, 8 * BBLK),
                                      jnp.float32),
        compiler_params=pltpu.CompilerParams(
            needs_layout_passes=False, use_tc_tiling_on_sc=False),
        scratch_types=[
            pltpu.VMEM((l_dim, BBLK), jnp.int32),
            pltpu.VMEM((BBLK, LANES), jnp.float32),
            pltpu.VMEM((BBLK, LANES), jnp.float32),
            pltpu.VMEM((n_hb, 8 * BBLK), jnp.float32),
            pltpu.VMEM((n_hb, 8 * BBLK), jnp.float32),
            pltpu.VMEM((LANES, LANES), jnp.float32),
            pltpu.SemaphoreType.DMA,
            pltpu.SemaphoreType.DMA,
            pltpu.SemaphoreType.DMA,
            pltpu.SemaphoreType.DMA,
        ],
    )
    def sc_encode(xt_hbm, emb_hbm, pos_hbm, out_hbm, idx_v, e_buf0, e_buf1,
                  o_buf0, o_buf1, pos_v, gsem0, gsem1, ssem0, ssem1):
        wid = lax.axis_index("s") * NC + lax.axis_index("c")
        e_bufs, o_bufs = (e_buf0, e_buf1), (o_buf0, o_buf1)
        gsems, ssems = (gsem0, gsem1), (ssem0, ssem1)
        pltpu.sync_copy(xt_hbm.at[:, pl.ds(wid * BBLK, BBLK)], idx_v)
        pltpu.sync_copy(pos_hbm, pos_v)
        riota = lax.iota(jnp.int32, LANES)
        jvs = [(riota + c) & (LANES - 1) for c in range(LANES)]
        hbs = [jv >> 3 for jv in jvs]
        obase = [(jv & 7) << 7 for jv in jvs]
        sixteen = jnp.full((LANES,), LANES, jnp.int32)
        nbuf = 2

        def gather_copy(l, b):
            return pltpu.make_async_copy(
                emb_hbm.at[idx_v.at[l]], e_bufs[b], gsems[b])

        def store_copies(l, b):
            return [
                pltpu.make_async_copy(
                    o_bufs[b].at[hb],
                    out_hbm.at[(l * n_hb + hb) * n_bt + wid], ssems[b])
                for hb in range(n_hb)
            ]

        for b in range(nbuf):
            gather_copy(b, b).start()

        def l_body(g, carry):
            for b in range(nbuf):
                l = g * nbuf + b
                gather_copy(l, b).wait()

                @pl.when(g > 0)
                def _():
                    for cp in store_copies(l - nbuf, b):
                        cp.wait()

                e_buf, o_buf = e_bufs[b], o_bufs[b]

                @plsc.parallel_loop(0, BBLK, step=LANES, unroll=2)
                def _(base):
                    rows = riota + base
                    vs = [plsc.load_gather(e_buf, [rows, jv]) for jv in jvs]
                    m = vs[0]
                    for c in range(1, LANES):
                        m = jnp.maximum(m, vs[c])
                    amax = sixteen
                    for c in range(LANES):
                        amax = jnp.minimum(
                            amax, jnp.where(vs[c] == m, jvs[c], sixteen))
                    for c in range(LANES):
                        p = plsc.load_gather(pos_v, [amax, jvs[c]])
                        plsc.store_scatter(
                            o_buf, [hbs[c], obase[c] + rows], vs[c] + p)

                for cp in store_copies(l, b):
                    cp.start()

                @pl.when(l + nbuf < l_dim)
                def _():
                    gather_copy(l + nbuf, b).start()
            return carry

        lax.fori_loop(0, l_dim // nbuf, l_body, 0)

        for b in range(nbuf):
            for cp in store_copies(l_dim - nbuf + b, b):
                cp.wait()

    return sc_encode


def _relayout_table(emb_table):
    v_dim, h_dim = emb_table.shape
    wb = 32768
    grid = pl.cdiv(v_dim, wb)

    def trans(in_ref, out_ref):
        v = in_ref[...].T.reshape(wb // 8, 8, LANES)
        out_ref[...] = jnp.concatenate([v[:, s, :] for s in range(8)], axis=1)

    y = pl.pallas_call(
        trans,
        grid=(grid,),
        in_specs=[pl.BlockSpec((h_dim, wb), lambda g: (0, g))],
        out_specs=pl.BlockSpec((wb // 8, 8 * LANES), lambda g: (g, 0)),
        out_shape=jax.ShapeDtypeStruct((v_dim * h_dim // (8 * LANES),
                                        8 * LANES), jnp.float32),
    )(emb_table.T)
    return y.reshape(v_dim, h_dim)


def kernel(x, emb_table, pos_table):
    b_dim, l_dim = x.shape
    _, h_dim = emb_table.shape
    call = _build_sc_call(b_dim, l_dim, h_dim)
    out = call(x.T.astype(jnp.int32), _relayout_table(emb_table), pos_table)
    n_hb, n_bt = h_dim // 8, b_dim // BBLK
    out = out.reshape(l_dim, n_hb, n_bt, 8, BBLK)
    return out.transpose(2, 4, 0, 1, 3).reshape(b_dim, l_dim, h_dim)


# speedup vs baseline: 2.5328x; 1.0498x over previous
import functools

import jax
import jax.numpy as jnp
from jax import lax
from jax.experimental import pallas as pl
from jax.experimental.pallas import tpu as pltpu
from jax.experimental.pallas import tpu_sc as plsc

NC = 2
NS = 16
NW = NC * NS
LANES = 16
BBLK = 128


def _build_sc_call(b_dim, l_dim, h_dim):
    assert h_dim == LANES and b_dim == NW * BBLK
    n_hb = h_dim // 8
    n_bt = b_dim // BBLK
    mesh = plsc.VectorSubcoreMesh(core_axis_name="c", subcore_axis_name="s")

    @functools.partial(
        pl.kernel,
        mesh=mesh,
        out_type=jax.ShapeDtypeStruct((l_dim * n_hb * n_bt, 8 * BBLK),
                                      jnp.float32),
        compiler_params=pltpu.CompilerParams(
            needs_layout_passes=False, use_tc_tiling_on_sc=False),
        scratch_types=[
            pltpu.VMEM((l_dim, BBLK), jnp.int32),
            pltpu.VMEM((BBLK, LANES), jnp.float32),
            pltpu.VMEM((BBLK, LANES), jnp.float32),
            pltpu.VMEM((n_hb, 8 * BBLK), jnp.float32),
            pltpu.VMEM((n_hb, 8 * BBLK), jnp.float32),
            pltpu.VMEM((LANES, LANES), jnp.float32),
            pltpu.SemaphoreType.DMA,
            pltpu.SemaphoreType.DMA,
            pltpu.SemaphoreType.DMA,
            pltpu.SemaphoreType.DMA,
        ],
    )
    def sc_encode(xt_hbm, emb_hbm, pos_hbm, out_hbm, idx_v, e_buf0, e_buf1,
                  o_buf0, o_buf1, pos_v, gsem0, gsem1, ssem0, ssem1):
        wid = lax.axis_index("s") * NC + lax.axis_index("c")
        e_bufs, o_bufs = (e_buf0, e_buf1), (o_buf0, o_buf1)
        gsems, ssems = (gsem0, gsem1), (ssem0, ssem1)
        pltpu.sync_copy(xt_hbm.at[:, pl.ds(wid * BBLK, BBLK)], idx_v)
        pltpu.sync_copy(pos_hbm, pos_v)
        riota = lax.iota(jnp.int32, LANES)
        jvs = [(riota + c) & (LANES - 1) for c in range(LANES)]
        hbs = [jv >> 3 for jv in jvs]
        obase = [(jv & 7) << 7 for jv in jvs]
        sixteen = jnp.full((LANES,), LANES, jnp.int32)
        nbuf = 2

        def gather_copy(l, b):
            return pltpu.make_async_copy(
                emb_hbm.at[idx_v.at[l]], e_bufs[b], gsems[b])

        def store_copies(l, b):
            return [
                pltpu.make_async_copy(
                    o_bufs[b].at[hb],
                    out_hbm.at[(l * n_hb + hb) * n_bt + wid], ssems[b])
                for hb in range(n_hb)
            ]

        for b in range(nbuf):
            gather_copy(b, b).start()

        def l_body(g, carry):
            for b in range(nbuf):
                l = g * nbuf + b
                gather_copy(l, b).wait()

                @pl.when(g > 0)
                def _():
                    for cp in store_copies(l - nbuf, b):
                        cp.wait()

                e_buf, o_buf = e_bufs[b], o_bufs[b]

                @plsc.parallel_loop(0, BBLK, step=LANES, unroll=2)
                def _(base):
                    rows = riota + base
                    vs = [plsc.load_gather(e_buf, [rows, jv]) for jv in jvs]
                    m = vs[0]
                    for c in range(1, LANES):
                        m = jnp.maximum(m, vs[c])
                    amax = sixteen
                    for c in range(LANES):
                        amax = jnp.minimum(
                            amax, jnp.where(vs[c] == m, jvs[c], sixteen))
                    for c in range(LANES):
                        p = plsc.load_gather(pos_v, [amax, jvs[c]])
                        plsc.store_scatter(
                            o_buf, [hbs[c], obase[c] + rows], vs[c] + p)

                for cp in store_copies(l, b):
                    cp.start()

                @pl.when(l + nbuf < l_dim)
                def _():
                    gather_copy(l + nbuf, b).start()
            return carry

        lax.fori_loop(0, l_dim // nbuf, l_body, 0)

        for b in range(nbuf):
            for cp in store_copies(l_dim - nbuf + b, b):
                cp.wait()

    return sc_encode


def _relayout_table(emb_table):
    v_dim, h_dim = emb_table.shape
    wb = 32768
    grid = pl.cdiv(v_dim, wb)

    def trans(in_ref, out_ref):
        eye = jnp.eye(LANES, dtype=jnp.float32)
        t = lax.dot_general(in_ref[...], eye, (((0,), (0,)), ((), ())),
                            preferred_element_type=jnp.float32)
        v = t.reshape(wb // 8, 8, LANES)
        out_ref[...] = jnp.concatenate([v[:, s, :] for s in range(8)], axis=1)

    y = pl.pallas_call(
        trans,
        grid=(grid,),
        in_specs=[pl.BlockSpec((h_dim, wb), lambda g: (0, g))],
        out_specs=pl.BlockSpec((wb // 8, 8 * LANES), lambda g: (g, 0)),
        out_shape=jax.ShapeDtypeStruct((v_dim * h_dim // (8 * LANES),
                                        8 * LANES), jnp.float32),
        compiler_params=pltpu.CompilerParams(
            dimension_semantics=("parallel",)),
    )(emb_table.T)
    return y.reshape(v_dim, h_dim)


def kernel(x, emb_table, pos_table):
    b_dim, l_dim = x.shape
    _, h_dim = emb_table.shape
    call = _build_sc_call(b_dim, l_dim, h_dim)
    out = call(x.T.astype(jnp.int32), _relayout_table(emb_table), pos_table)
    n_hb, n_bt = h_dim // 8, b_dim // BBLK
    out = out.reshape(l_dim, n_hb, n_bt, 8, BBLK)
    return out.transpose(2, 4, 0, 1, 3).reshape(b_dim, l_dim, h_dim)
